# baseline (speedup 1.0000x reference)
import functools

import jax
import jax.numpy as jnp
from jax import lax
from jax.experimental import pallas as pl
from jax.experimental.pallas import tpu as pltpu
from jax.experimental.pallas import tpu_sc as plsc

B = 16384
L = 50
D = 16
NC = 2
NS = 16
NW = NC * NS
BPW = B // NW
GRP = BPW // 16


def _emb_body(user_id, item_id, cat_t, brand, w_user_t, w_item_t,
              w_category, w_brand, out_hbm,
              uidx, iidx, bidx, cols, u_t, i_t, c_t, b_t, brows,
              acc, n0, w0,
              sem_u, sem_i, sem_b, sem_c):
    wid = lax.axis_index("s") * NC + lax.axis_index("c")
    base = wid * BPW

    pltpu.sync_copy(user_id.at[pl.ds(base, BPW)], uidx)
    pltpu.sync_copy(item_id.at[pl.ds(base, BPW)], iidx)
    pltpu.sync_copy(brand.at[pl.ds(base, BPW)], bidx)
    pltpu.sync_copy(cat_t.at[:, pl.ds(base, BPW)], cols)

    for d in range(D):
        pltpu.async_copy(w_user_t.at[d].at[uidx], u_t.at[d], sem_u)
        pltpu.async_copy(w_item_t.at[d].at[iidx], i_t.at[d], sem_i)
    db = pltpu.async_copy(w_brand.at[bidx], brows, sem_b)

    lane = jnp.arange(16, dtype=jnp.int32)

    def g_body(g, _):
        def l_body(l, cnt):
            vals = cols[l, pl.ds(g * 16, 16)]
            return cnt + jnp.where(vals == 0, 1, 0).astype(jnp.int32)

        cnt = lax.fori_loop(0, L, l_body, jnp.zeros((16,), jnp.int32))
        n0[pl.ds(g * 16, 16)] = cnt.astype(jnp.float32)
        return 0

    lax.fori_loop(0, GRP, g_body, 0)

    pltpu.async_copy(w_category.at[cols.at[0]], acc, sem_c).wait()

    def c_fire(l, _):
        pltpu.async_copy(w_category.at[cols.at[l]], acc, sem_c, add=True)
        return 0

    lax.fori_loop(1, L, c_fire, 0)

    def c_drain(l, _):
        pltpu.make_async_copy(w_category.at[cols.at[0]], acc, sem_c).wait()
        return 0

    lax.fori_loop(1, L, c_drain, 0)

    pltpu.sync_copy(w_category.at[pl.ds(0, 1)], w0)

    db.wait()

    inv_l = jnp.float32(1.0 / L)
    w0s = w0[0, :] * inv_l
    lane_c = lane * BPW

    def o_body(b, _):
        n0b = plsc.load_gather(n0, [jnp.full((16,), b, jnp.int32)])
        cv = acc[b, :] * inv_l - n0b * w0s
        plsc.store_scatter(c_t, [lane, jnp.full((16,), b, jnp.int32)], cv)
        plsc.store_scatter(b_t, [lane, jnp.full((16,), b, jnp.int32)], brows[b, :])
        return 0

    lax.fori_loop(0, BPW, o_body, 0)

    for d in range(D):
        pltpu.make_async_copy(w_user_t.at[0].at[uidx], u_t.at[0], sem_u).wait()
        pltpu.make_async_copy(w_item_t.at[0].at[iidx], i_t.at[0], sem_i).wait()

    pltpu.sync_copy(u_t, out_hbm.at[0, :, pl.ds(base, BPW)])
    pltpu.sync_copy(i_t, out_hbm.at[1, :, pl.ds(base, BPW)])
    pltpu.sync_copy(c_t, out_hbm.at[2, :, pl.ds(base, BPW)])
    pltpu.sync_copy(b_t, out_hbm.at[3, :, pl.ds(base, BPW)])


@jax.jit
def _emb(user_id, item_id, category, brand, w_user, w_item, w_category, w_brand):
    mesh = plsc.VectorSubcoreMesh(core_axis_name="c", subcore_axis_name="s")
    f = pl.kernel(
        _emb_body,
        out_type=jax.ShapeDtypeStruct((4, D, B), jnp.float32),
        mesh=mesh,
        compiler_params=pltpu.CompilerParams(
            needs_layout_passes=False, use_tc_tiling_on_sc=False),
        scratch_types=[
            pltpu.VMEM((BPW,), jnp.int32),
            pltpu.VMEM((BPW,), jnp.int32),
            pltpu.VMEM((BPW,), jnp.int32),
            pltpu.VMEM((L, BPW), jnp.int32),
            pltpu.VMEM((D, BPW), jnp.float32),
            pltpu.VMEM((D, BPW), jnp.float32),
            pltpu.VMEM((D, BPW), jnp.float32),
            pltpu.VMEM((D, BPW), jnp.float32),
            pltpu.VMEM((BPW, D), jnp.float32),
            pltpu.VMEM((BPW, D), jnp.float32),
            pltpu.VMEM((BPW,), jnp.float32),
            pltpu.VMEM((1, D), jnp.float32),
            pltpu.SemaphoreType.DMA,
            pltpu.SemaphoreType.DMA,
            pltpu.SemaphoreType.DMA,
            pltpu.SemaphoreType.DMA,
        ],
    )
    out = f(user_id, item_id, category.T, brand,
            w_user.T, w_item.T, w_category, w_brand)
    return jnp.transpose(out, (2, 0, 1))


def kernel(label, user_id, item_id, category, brand,
           W_user, W_item, W_category, W_brand):
    del label
    return _emb(user_id, item_id, category, brand,
                W_user, W_item, W_category, W_brand)


# speedup vs baseline: 3.0597x; 3.0597x over previous
import functools

import jax
import jax.numpy as jnp
from jax import lax
from jax.experimental import pallas as pl
from jax.experimental.pallas import tpu as pltpu
from jax.experimental.pallas import tpu_sc as plsc

B = 16384
L = 50
D = 16
NC = 2
NS = 16
NW = NC * NS
BPW = B // NW
GRP = BPW // 16


def _emb_body(user_id, item_id, cat_t, brand, w_user, w_item,
              w_category, w_brand, out_hbm,
              uidx, iidx, bidx, cols, urows, irows, brows,
              u_t, i_t, c_t, b_t, acc, n0, w0,
              sem_u, sem_i, sem_b, sem_c):
    wid = lax.axis_index("s") * NC + lax.axis_index("c")
    base = wid * BPW

    pltpu.sync_copy(user_id.at[pl.ds(base, BPW)], uidx)
    pltpu.sync_copy(item_id.at[pl.ds(base, BPW)], iidx)
    pltpu.sync_copy(brand.at[pl.ds(base, BPW)], bidx)
    pltpu.sync_copy(cat_t.at[:, pl.ds(base, BPW)], cols)

    du = pltpu.async_copy(w_user.at[uidx], urows, sem_u)
    di = pltpu.async_copy(w_item.at[iidx], irows, sem_i)
    db = pltpu.async_copy(w_brand.at[bidx], brows, sem_b)

    pltpu.async_copy(w_category.at[cols.at[0]], acc, sem_c).wait()

    def c_fire(l, _):
        pltpu.async_copy(w_category.at[cols.at[l]], acc, sem_c, add=True)
        return 0

    lax.fori_loop(1, L, c_fire, 0)

    lane = jnp.arange(16, dtype=jnp.int32)

    def g_body(g, _):
        def l_body(l, cnt):
            vals = cols[l, pl.ds(g * 16, 16)]
            return cnt + jnp.where(vals == 0, 1, 0).astype(jnp.int32)

        cnt = lax.fori_loop(0, L, l_body, jnp.zeros((16,), jnp.int32))
        n0[pl.ds(g * 16, 16)] = cnt.astype(jnp.float32)
        return 0

    lax.fori_loop(0, GRP, g_body, 0)

    pltpu.sync_copy(w_category.at[pl.ds(0, 1)], w0)

    def c_drain(l, _):
        pltpu.make_async_copy(w_category.at[cols.at[0]], acc, sem_c).wait()
        return 0

    lax.fori_loop(1, L, c_drain, 0)
    du.wait()
    di.wait()
    db.wait()

    inv_l = jnp.float32(1.0 / L)
    w0s = w0[0, :] * inv_l

    def o_body(b, _):
        bvec = jnp.full((16,), b, jnp.int32)
        n0b = plsc.load_gather(n0, [bvec])
        cv = acc[b, :] * inv_l - n0b * w0s
        plsc.store_scatter(c_t, [lane, bvec], cv)
        plsc.store_scatter(u_t, [lane, bvec], urows[b, :])
        plsc.store_scatter(i_t, [lane, bvec], irows[b, :])
        plsc.store_scatter(b_t, [lane, bvec], brows[b, :])
        return 0

    lax.fori_loop(0, BPW, o_body, 0)

    pltpu.sync_copy(u_t, out_hbm.at[0, :, pl.ds(base, BPW)])
    pltpu.sync_copy(i_t, out_hbm.at[1, :, pl.ds(base, BPW)])
    pltpu.sync_copy(c_t, out_hbm.at[2, :, pl.ds(base, BPW)])
    pltpu.sync_copy(b_t, out_hbm.at[3, :, pl.ds(base, BPW)])


@jax.jit
def _emb(user_id, item_id, category, brand, w_user, w_item, w_category, w_brand):
    mesh = plsc.VectorSubcoreMesh(core_axis_name="c", subcore_axis_name="s")
    f = pl.kernel(
        _emb_body,
        out_type=jax.ShapeDtypeStruct((4, D, B), jnp.float32),
        mesh=mesh,
        compiler_params=pltpu.CompilerParams(
            needs_layout_passes=False, use_tc_tiling_on_sc=False),
        scratch_types=[
            pltpu.VMEM((BPW,), jnp.int32),
            pltpu.VMEM((BPW,), jnp.int32),
            pltpu.VMEM((BPW,), jnp.int32),
            pltpu.VMEM((L, BPW), jnp.int32),
            pltpu.VMEM((BPW, D), jnp.float32),
            pltpu.VMEM((BPW, D), jnp.float32),
            pltpu.VMEM((BPW, D), jnp.float32),
            pltpu.VMEM((D, BPW), jnp.float32),
            pltpu.VMEM((D, BPW), jnp.float32),
            pltpu.VMEM((D, BPW), jnp.float32),
            pltpu.VMEM((D, BPW), jnp.float32),
            pltpu.VMEM((BPW, D), jnp.float32),
            pltpu.VMEM((BPW,), jnp.float32),
            pltpu.VMEM((1, D), jnp.float32),
            pltpu.SemaphoreType.DMA,
            pltpu.SemaphoreType.DMA,
            pltpu.SemaphoreType.DMA,
            pltpu.SemaphoreType.DMA,
        ],
    )
    out = f(user_id, item_id, category.T, brand,
            w_user, w_item, w_category, w_brand)
    return jnp.transpose(out, (2, 0, 1))


def kernel(label, user_id, item_id, category, brand,
           W_user, W_item, W_category, W_brand):
    del label
    return _emb(user_id, item_id, category, brand,
                W_user, W_item, W_category, W_brand)


# speedup vs baseline: 12.5767x; 4.1104x over previous
import functools

import jax
import jax.numpy as jnp
from jax import lax
from jax.experimental import pallas as pl
from jax.experimental.pallas import tpu as pltpu
from jax.experimental.pallas import tpu_sc as plsc

B = 16384
L = 50
D = 16
NC = 2
NS = 16
NW = NC * NS
BPW = B // NW
GRP = BPW // 16

V_ID = 1000001
V_SM = 100001
VP_ID = 1000064
VP_SM = 100096
CT_ID = VP_ID // 128
CT_SM = VP_SM // 128


def _addr_build(idxv, addrs, ct):
    offs = [((d // 8) * ct * 8 + d % 8) * 128 for d in range(D)]

    def g_body(g, _):
        ids = idxv[pl.ds(g * 16, 16)]
        bse = ((ids >> 7) << 10) + (ids & 127)
        for d in range(D):
            addrs[pl.ds(d * BPW + g * 16, 16)] = bse + offs[d]
        return 0

    lax.fori_loop(0, GRP, g_body, 0)


def _emb_body(user_id, item_id, cat_t, brand, w_user_f, w_item_f,
              w_category, w_brand_f, out_hbm,
              uidx, iidx, bidx, cols, au, ai, ab,
              u_t, i_t, c_t, b_t, acc, n0, w0,
              sem_u, sem_i, sem_b, sem_c):
    wid = lax.axis_index("s") * NC + lax.axis_index("c")
    base = wid * BPW

    pltpu.sync_copy(user_id.at[pl.ds(base, BPW)], uidx)
    pltpu.sync_copy(item_id.at[pl.ds(base, BPW)], iidx)
    pltpu.sync_copy(brand.at[pl.ds(base, BPW)], bidx)
    pltpu.sync_copy(cat_t.at[:, pl.ds(base, BPW)], cols)

    pltpu.async_copy(w_category.at[cols.at[0]], acc, sem_c).wait()

    def c_fire(l, _):
        pltpu.async_copy(w_category.at[cols.at[l]], acc, sem_c, add=True)
        return 0

    lax.fori_loop(1, L, c_fire, 0)

    _addr_build(uidx, au, CT_ID)
    _addr_build(iidx, ai, CT_ID)
    _addr_build(bidx, ab, CT_SM)

    for d in range(D):
        pltpu.async_copy(w_user_f.at[au.at[pl.ds(d * BPW, BPW)]], u_t.at[d], sem_u)
        pltpu.async_copy(w_item_f.at[ai.at[pl.ds(d * BPW, BPW)]], i_t.at[d], sem_i)
        pltpu.async_copy(w_brand_f.at[ab.at[pl.ds(d * BPW, BPW)]], b_t.at[d], sem_b)

    def g_body(g, _):
        def l_body(l, cnt):
            vals = cols[l, pl.ds(g * 16, 16)]
            return cnt + jnp.where(vals == 0, 1, 0).astype(jnp.int32)

        cnt = lax.fori_loop(0, L, l_body, jnp.zeros((16,), jnp.int32))
        n0[pl.ds(g * 16, 16)] = cnt.astype(jnp.float32)
        return 0

    lax.fori_loop(0, GRP, g_body, 0)

    pltpu.sync_copy(w_category.at[pl.ds(0, 1)], w0)

    def c_drain(l, _):
        pltpu.make_async_copy(w_category.at[cols.at[0]], acc, sem_c).wait()
        return 0

    lax.fori_loop(1, L, c_drain, 0)

    inv_l = jnp.float32(1.0 / L)
    w0s = w0[0, :] * inv_l
    lane = jnp.arange(16, dtype=jnp.int32)

    def o_body(b, _):
        bvec = jnp.full((16,), b, jnp.int32)
        n0b = plsc.load_gather(n0, [bvec])
        cv = acc[b, :] * inv_l - n0b * w0s
        plsc.store_scatter(c_t, [lane, bvec], cv)
        return 0

    lax.fori_loop(0, BPW, o_body, 0)

    for d in range(D):
        pltpu.make_async_copy(w_user_f.at[au.at[pl.ds(0, BPW)]], u_t.at[0], sem_u).wait()
        pltpu.make_async_copy(w_item_f.at[ai.at[pl.ds(0, BPW)]], i_t.at[0], sem_i).wait()
        pltpu.make_async_copy(w_brand_f.at[ab.at[pl.ds(0, BPW)]], b_t.at[0], sem_b).wait()

    pltpu.sync_copy(u_t, out_hbm.at[0, :, pl.ds(base, BPW)])
    pltpu.sync_copy(i_t, out_hbm.at[1, :, pl.ds(base, BPW)])
    pltpu.sync_copy(c_t, out_hbm.at[2, :, pl.ds(base, BPW)])
    pltpu.sync_copy(b_t, out_hbm.at[3, :, pl.ds(base, BPW)])


def _tiled_flat(w, vp):
    v = w.shape[0]
    wp = jnp.pad(w.T, ((0, 0), (0, vp - v)))
    return wp.reshape(2, 8, vp // 128, 128).transpose(0, 2, 1, 3).reshape(-1)


@jax.jit
def _emb(user_id, item_id, category, brand, w_user, w_item, w_category, w_brand):
    mesh = plsc.VectorSubcoreMesh(core_axis_name="c", subcore_axis_name="s")
    f = pl.kernel(
        _emb_body,
        out_type=jax.ShapeDtypeStruct((4, D, B), jnp.float32),
        mesh=mesh,
        compiler_params=pltpu.CompilerParams(
            needs_layout_passes=False, use_tc_tiling_on_sc=False),
        scratch_types=[
            pltpu.VMEM((BPW,), jnp.int32),
            pltpu.VMEM((BPW,), jnp.int32),
            pltpu.VMEM((BPW,), jnp.int32),
            pltpu.VMEM((L, BPW), jnp.int32),
            pltpu.VMEM((D * BPW,), jnp.int32),
            pltpu.VMEM((D * BPW,), jnp.int32),
            pltpu.VMEM((D * BPW,), jnp.int32),
            pltpu.VMEM((D, BPW), jnp.float32),
            pltpu.VMEM((D, BPW), jnp.float32),
            pltpu.VMEM((D, BPW), jnp.float32),
            pltpu.VMEM((D, BPW), jnp.float32),
            pltpu.VMEM((BPW, D), jnp.float32),
            pltpu.VMEM((BPW,), jnp.float32),
            pltpu.VMEM((1, D), jnp.float32),
            pltpu.SemaphoreType.DMA,
            pltpu.SemaphoreType.DMA,
            pltpu.SemaphoreType.DMA,
            pltpu.SemaphoreType.DMA,
        ],
    )
    out = f(user_id, item_id, category.T, brand,
            _tiled_flat(w_user, VP_ID), _tiled_flat(w_item, VP_ID),
            w_category, _tiled_flat(w_brand, VP_SM))
    return jnp.transpose(out, (2, 0, 1))


def kernel(label, user_id, item_id, category, brand,
           W_user, W_item, W_category, W_brand):
    del label
    return _emb(user_id, item_id, category, brand,
                W_user, W_item, W_category, W_brand)


# speedup vs baseline: 14.1200x; 1.1227x over previous
import functools

import jax
import jax.numpy as jnp
from jax import lax
from jax.experimental import pallas as pl
from jax.experimental.pallas import tpu as pltpu
from jax.experimental.pallas import tpu_sc as plsc

B = 16384
L = 50
D = 16
NC = 2
NS = 16
NW = NC * NS
BPW = B // NW
GRP = BPW // 16

V_ID = 1000001
V_SM = 100001
VP_ID = 1000064
VP_SM = 100096
CT_ID = VP_ID // 128
CT_SM = VP_SM // 128

_COMPILER_PARAMS = pltpu.CompilerParams(
    needs_layout_passes=False, use_tc_tiling_on_sc=False)


def _cat_body(cat_t, w_category, out_hbm, cols, acc, n0, w0, c_t, sem_c):
    wid = lax.axis_index("s") * NC + lax.axis_index("c")
    base = wid * BPW

    pltpu.sync_copy(cat_t.at[:, pl.ds(base, BPW)], cols)

    pltpu.async_copy(w_category.at[cols.at[0]], acc, sem_c).wait()

    def c_fire(l, _):
        pltpu.async_copy(w_category.at[cols.at[l]], acc, sem_c, add=True)
        return 0

    lax.fori_loop(1, L, c_fire, 0)

    def g_body(g, _):
        def l_body(l, cnt):
            vals = cols[l, pl.ds(g * 16, 16)]
            return cnt + jnp.where(vals == 0, 1, 0).astype(jnp.int32)

        cnt = lax.fori_loop(0, L, l_body, jnp.zeros((16,), jnp.int32))
        n0[pl.ds(g * 16, 16)] = cnt.astype(jnp.float32)
        return 0

    lax.fori_loop(0, GRP, g_body, 0)

    pltpu.sync_copy(w_category.at[pl.ds(0, 1)], w0)

    def c_drain(l, _):
        pltpu.make_async_copy(w_category.at[cols.at[0]], acc, sem_c).wait()
        return 0

    lax.fori_loop(1, L, c_drain, 0)

    inv_l = jnp.float32(1.0 / L)
    w0s = w0[0, :] * inv_l
    lane = jnp.arange(16, dtype=jnp.int32)

    def o_body(b, _):
        bvec = jnp.full((16,), b, jnp.int32)
        n0b = plsc.load_gather(n0, [bvec])
        cv = acc[b, :] * inv_l - n0b * w0s
        plsc.store_scatter(c_t, [lane, bvec], cv)
        return 0

    lax.fori_loop(0, BPW, o_body, 0)

    pltpu.sync_copy(c_t, out_hbm.at[:, pl.ds(base, BPW)])


def _addr_build(idxv, addrs, ct):
    offs = [((d // 8) * ct * 8 + d % 8) * 128 for d in range(D)]

    def g_body(g, _):
        ids = idxv[pl.ds(g * 16, 16)]
        bse = ((ids >> 7) << 10) + (ids & 127)
        for d in range(D):
            addrs[pl.ds(d * BPW + g * 16, 16)] = bse + offs[d]
        return 0

    lax.fori_loop(0, GRP, g_body, 0)


def _uib_body(user_id, item_id, brand, w_user_f, w_item_f, w_brand_f,
              out_hbm, uidx, iidx, bidx, au, ai, ab, u_t, i_t, b_t,
              sem_u, sem_i, sem_b):
    wid = lax.axis_index("s") * NC + lax.axis_index("c")
    base = wid * BPW

    pltpu.sync_copy(user_id.at[pl.ds(base, BPW)], uidx)
    pltpu.sync_copy(item_id.at[pl.ds(base, BPW)], iidx)
    pltpu.sync_copy(brand.at[pl.ds(base, BPW)], bidx)

    _addr_build(uidx, au, CT_ID)
    _addr_build(iidx, ai, CT_ID)
    _addr_build(bidx, ab, CT_SM)

    for d in range(D):
        pltpu.async_copy(w_user_f.at[au.at[pl.ds(d * BPW, BPW)]], u_t.at[d], sem_u)
        pltpu.async_copy(w_item_f.at[ai.at[pl.ds(d * BPW, BPW)]], i_t.at[d], sem_i)
        pltpu.async_copy(w_brand_f.at[ab.at[pl.ds(d * BPW, BPW)]], b_t.at[d], sem_b)
    for d in range(D):
        pltpu.make_async_copy(w_user_f.at[au.at[pl.ds(0, BPW)]], u_t.at[0], sem_u).wait()
        pltpu.make_async_copy(w_item_f.at[ai.at[pl.ds(0, BPW)]], i_t.at[0], sem_i).wait()
        pltpu.make_async_copy(w_brand_f.at[ab.at[pl.ds(0, BPW)]], b_t.at[0], sem_b).wait()

    pltpu.sync_copy(u_t, out_hbm.at[0, :, pl.ds(base, BPW)])
    pltpu.sync_copy(i_t, out_hbm.at[1, :, pl.ds(base, BPW)])
    pltpu.sync_copy(b_t, out_hbm.at[2, :, pl.ds(base, BPW)])


def _tiled_flat(w, vp):
    v = w.shape[0]
    wp = jnp.pad(w.T, ((0, 0), (0, vp - v)))
    return wp.reshape(2, 8, vp // 128, 128).transpose(0, 2, 1, 3).reshape(-1)


@jax.jit
def _emb(user_id, item_id, category, brand, w_user, w_item, w_category, w_brand):
    mesh = plsc.VectorSubcoreMesh(core_axis_name="c", subcore_axis_name="s")

    cat_k = pl.kernel(
        _cat_body,
        out_type=jax.ShapeDtypeStruct((D, B), jnp.float32),
        mesh=mesh,
        compiler_params=_COMPILER_PARAMS,
        scratch_types=[
            pltpu.VMEM((L, BPW), jnp.int32),
            pltpu.VMEM((BPW, D), jnp.float32),
            pltpu.VMEM((BPW,), jnp.float32),
            pltpu.VMEM((1, D), jnp.float32),
            pltpu.VMEM((D, BPW), jnp.float32),
            pltpu.SemaphoreType.DMA,
        ],
    )
    c_pl = cat_k(category.T, w_category)

    uib_k = pl.kernel(
        _uib_body,
        out_type=jax.ShapeDtypeStruct((3, D, B), jnp.float32),
        mesh=mesh,
        compiler_params=_COMPILER_PARAMS,
        scratch_types=[
            pltpu.VMEM((BPW,), jnp.int32),
            pltpu.VMEM((BPW,), jnp.int32),
            pltpu.VMEM((BPW,), jnp.int32),
            pltpu.VMEM((D * BPW,), jnp.int32),
            pltpu.VMEM((D * BPW,), jnp.int32),
            pltpu.VMEM((D * BPW,), jnp.int32),
            pltpu.VMEM((D, BPW), jnp.float32),
            pltpu.VMEM((D, BPW), jnp.float32),
            pltpu.VMEM((D, BPW), jnp.float32),
            pltpu.SemaphoreType.DMA,
            pltpu.SemaphoreType.DMA,
            pltpu.SemaphoreType.DMA,
        ],
    )
    uib = uib_k(user_id, item_id, brand,
                _tiled_flat(w_user, VP_ID), _tiled_flat(w_item, VP_ID),
                _tiled_flat(w_brand, VP_SM))

    out = jnp.concatenate(
        [uib[0:2], c_pl[None], uib[2:3]], axis=0)
    return jnp.transpose(out, (2, 0, 1))


def kernel(label, user_id, item_id, category, brand,
           W_user, W_item, W_category, W_brand):
    del label
    return _emb(user_id, item_id, category, brand,
                W_user, W_item, W_category, W_brand)


# speedup vs baseline: 14.4462x; 1.0231x over previous
import functools

import jax
import jax.numpy as jnp
from jax import lax
from jax.experimental import pallas as pl
from jax.experimental.pallas import tpu as pltpu
from jax.experimental.pallas import tpu_sc as plsc

B = 16384
L = 50
D = 16
NC = 2
NS = 16
NW = NC * NS
BPW = B // NW
GRP = BPW // 16

V_ID = 1000001
V_SM = 100001
VP_ID = 1000064
VP_SM = 100096
CT_ID = VP_ID // 128
CT_SM = VP_SM // 128

_COMPILER_PARAMS = pltpu.CompilerParams(
    needs_layout_passes=False, use_tc_tiling_on_sc=False)

VP_CAT = 102400
CTILES = VP_CAT // 128
TPW = CTILES // NW


def _detile_body(w_cat_f, out_hbm, chunk, rows):
    wid = lax.axis_index("s") * NC + lax.axis_index("c")
    tc0 = wid * TPW

    pltpu.sync_copy(w_cat_f.at[pl.ds(tc0 * 1024, TPW * 1024)],
                    chunk.at[pl.ds(0, TPW * 1024)])
    pltpu.sync_copy(w_cat_f.at[pl.ds((CTILES + tc0) * 1024, TPW * 1024)],
                    chunk.at[pl.ds(TPW * 1024, TPW * 1024)])

    lane = jnp.arange(16, dtype=jnp.int32)
    loff = jnp.where(lane < 8, lane * 128, TPW * 1024 + (lane - 8) * 128)

    def t_body(t, _):
        def j_body(j, _):
            idx = loff + (t * 1024 + j)
            vals = plsc.load_gather(chunk, [idx])
            rows[t * 128 + j, :] = vals
            return 0

        lax.fori_loop(0, 128, j_body, 0)
        return 0

    lax.fori_loop(0, TPW, t_body, 0)

    pltpu.sync_copy(rows, out_hbm.at[pl.ds(tc0 * 128, TPW * 128)])


def _cat_body(cat_t, w_category, out_hbm, cols, acc, n0, w0, c_t, sem_c):
    wid = lax.axis_index("s") * NC + lax.axis_index("c")
    base = wid * BPW

    pltpu.sync_copy(cat_t.at[:, pl.ds(base, BPW)], cols)

    pltpu.async_copy(w_category.at[cols.at[0]], acc, sem_c).wait()

    def c_fire(l, _):
        pltpu.async_copy(w_category.at[cols.at[l]], acc, sem_c, add=True)
        return 0

    lax.fori_loop(1, L, c_fire, 0)

    def g_body(g, _):
        def l_body(l, cnt):
            vals = cols[l, pl.ds(g * 16, 16)]
            return cnt + jnp.where(vals == 0, 1, 0).astype(jnp.int32)

        cnt = lax.fori_loop(0, L, l_body, jnp.zeros((16,), jnp.int32))
        n0[pl.ds(g * 16, 16)] = cnt.astype(jnp.float32)
        return 0

    lax.fori_loop(0, GRP, g_body, 0)

    pltpu.sync_copy(w_category.at[pl.ds(0, 1)], w0)

    def c_drain(l, _):
        pltpu.make_async_copy(w_category.at[cols.at[0]], acc, sem_c).wait()
        return 0

    lax.fori_loop(1, L, c_drain, 0)

    inv_l = jnp.float32(1.0 / L)
    w0s = w0[0, :] * inv_l
    lane = jnp.arange(16, dtype=jnp.int32)

    def o_body(b, _):
        bvec = jnp.full((16,), b, jnp.int32)
        n0b = plsc.load_gather(n0, [bvec])
        cv = acc[b, :] * inv_l - n0b * w0s
        plsc.store_scatter(c_t, [lane, bvec], cv)
        return 0

    lax.fori_loop(0, BPW, o_body, 0)

    pltpu.sync_copy(c_t, out_hbm.at[:, pl.ds(base, BPW)])


def _addr_build(idxv, addrs, ct):
    offs = [((d // 8) * ct * 8 + d % 8) * 128 for d in range(D)]

    def g_body(g, _):
        ids = idxv[pl.ds(g * 16, 16)]
        bse = ((ids >> 7) << 10) + (ids & 127)
        for d in range(D):
            addrs[pl.ds(d * BPW + g * 16, 16)] = bse + offs[d]
        return 0

    lax.fori_loop(0, GRP, g_body, 0)


def _uib_body(user_id, item_id, brand, w_user_f, w_item_f, w_brand_f,
              out_hbm, uidx, iidx, bidx, au, ai, ab, u_t, i_t, b_t,
              sem_u, sem_i, sem_b):
    wid = lax.axis_index("s") * NC + lax.axis_index("c")
    base = wid * BPW

    pltpu.sync_copy(user_id.at[pl.ds(base, BPW)], uidx)
    pltpu.sync_copy(item_id.at[pl.ds(base, BPW)], iidx)
    pltpu.sync_copy(brand.at[pl.ds(base, BPW)], bidx)

    _addr_build(uidx, au, CT_ID)
    _addr_build(iidx, ai, CT_ID)
    _addr_build(bidx, ab, CT_SM)

    for d in range(D):
        pltpu.async_copy(w_user_f.at[au.at[pl.ds(d * BPW, BPW)]], u_t.at[d], sem_u)
        pltpu.async_copy(w_item_f.at[ai.at[pl.ds(d * BPW, BPW)]], i_t.at[d], sem_i)
        pltpu.async_copy(w_brand_f.at[ab.at[pl.ds(d * BPW, BPW)]], b_t.at[d], sem_b)
    for d in range(D):
        pltpu.make_async_copy(w_user_f.at[au.at[pl.ds(0, BPW)]], u_t.at[0], sem_u).wait()
        pltpu.make_async_copy(w_item_f.at[ai.at[pl.ds(0, BPW)]], i_t.at[0], sem_i).wait()
        pltpu.make_async_copy(w_brand_f.at[ab.at[pl.ds(0, BPW)]], b_t.at[0], sem_b).wait()

    pltpu.sync_copy(u_t, out_hbm.at[0, :, pl.ds(base, BPW)])
    pltpu.sync_copy(i_t, out_hbm.at[1, :, pl.ds(base, BPW)])
    pltpu.sync_copy(b_t, out_hbm.at[2, :, pl.ds(base, BPW)])


def _tiled_flat(w, vp):
    v = w.shape[0]
    wp = jnp.pad(w.T, ((0, 0), (0, vp - v)))
    return wp.reshape(2, 8, vp // 128, 128).transpose(0, 2, 1, 3).reshape(-1)


@jax.jit
def _emb(user_id, item_id, category, brand, w_user, w_item, w_category, w_brand):
    mesh = plsc.VectorSubcoreMesh(core_axis_name="c", subcore_axis_name="s")

    detile_k = pl.kernel(
        _detile_body,
        out_type=jax.ShapeDtypeStruct((VP_CAT, D), jnp.float32),
        mesh=mesh,
        compiler_params=_COMPILER_PARAMS,
        scratch_types=[
            pltpu.VMEM((2 * TPW * 1024,), jnp.float32),
            pltpu.VMEM((TPW * 128, D), jnp.float32),
        ],
    )
    w_cat_rows = detile_k(_tiled_flat(w_category, VP_CAT))

    cat_k = pl.kernel(
        _cat_body,
        out_type=jax.ShapeDtypeStruct((D, B), jnp.float32),
        mesh=mesh,
        compiler_params=_COMPILER_PARAMS,
        scratch_types=[
            pltpu.VMEM((L, BPW), jnp.int32),
            pltpu.VMEM((BPW, D), jnp.float32),
            pltpu.VMEM((BPW,), jnp.float32),
            pltpu.VMEM((1, D), jnp.float32),
            pltpu.VMEM((D, BPW), jnp.float32),
            pltpu.SemaphoreType.DMA,
        ],
    )
    c_pl = cat_k(category.T, w_cat_rows)

    uib_k = pl.kernel(
        _uib_body,
        out_type=jax.ShapeDtypeStruct((3, D, B), jnp.float32),
        mesh=mesh,
        compiler_params=_COMPILER_PARAMS,
        scratch_types=[
            pltpu.VMEM((BPW,), jnp.int32),
            pltpu.VMEM((BPW,), jnp.int32),
            pltpu.VMEM((BPW,), jnp.int32),
            pltpu.VMEM((D * BPW,), jnp.int32),
            pltpu.VMEM((D * BPW,), jnp.int32),
            pltpu.VMEM((D * BPW,), jnp.int32),
            pltpu.VMEM((D, BPW), jnp.float32),
            pltpu.VMEM((D, BPW), jnp.float32),
            pltpu.VMEM((D, BPW), jnp.float32),
            pltpu.SemaphoreType.DMA,
            pltpu.SemaphoreType.DMA,
            pltpu.SemaphoreType.DMA,
        ],
    )
    uib = uib_k(user_id, item_id, brand,
                _tiled_flat(w_user, VP_ID), _tiled_flat(w_item, VP_ID),
                _tiled_flat(w_brand, VP_SM))

    out = jnp.concatenate(
        [uib[0:2], c_pl[None], uib[2:3]], axis=0)
    return jnp.transpose(out, (2, 0, 1))


def kernel(label, user_id, item_id, category, brand,
           W_user, W_item, W_category, W_brand):
    del label
    return _emb(user_id, item_id, category, brand,
                W_user, W_item, W_category, W_brand)
